# Initial kernel scaffold; baseline (speedup 1.0000x reference)
import jax
import jax.numpy as jnp
from jax.experimental import pallas as pl


def kernel(X0, X1, X2, L0, L1, L2, batch0, batch1, batch2, W01, b01, W02, b02, W03, b03, W11, b11, W12, b12, W13, b13, W21, b21, W22, b22, W23, b23):
    raise NotImplementedError("write your pallas kernel here")


# speedup vs baseline: 1.4276x; 1.4276x over previous
import jax
import jax.numpy as jnp
from jax.experimental import pallas as pl

N = 4096
H = 32
OUT = 64
BLK = 256
NBLK = N // BLK


def _leaky(x):
    return jnp.where(x > 0, x, 0.01 * x)


def _pre_body(x0, x1, x2, w1s, y_ref):
    for b, xr in enumerate((x0, x1, x2)):
        y_ref[b] = jnp.dot(_leaky(xr[...]), w1s[b],
                           preferred_element_type=jnp.float32)


def _main_body(l0, l1, l2, y, b1s, a_ref, c_ref):
    i = pl.program_id(0)
    for b, lr in enumerate((l0, l1, l2)):
        lb = lr[...]
        z = jnp.dot(lb, y[b], preferred_element_type=jnp.float32)
        a_ref[b] = _leaky(z + b1s[b][None, :])
        csum = jnp.sum(lb, axis=0, keepdims=True)

        @pl.when(i == 0)
        def _():
            c_ref[b] = csum

        @pl.when(i != 0)
        def _():
            c_ref[b] += csum


def _v_body(l0, l1, l2, c, v_ref):
    i = pl.program_id(0)
    for b, lr in enumerate((l0, l1, l2)):
        part = jnp.dot(c[b], lr[...],
                       preferred_element_type=jnp.float32)

        @pl.when(i == 0)
        def _():
            v_ref[b] = part

        @pl.when(i != 0)
        def _():
            v_ref[b] += part


def _final_body(a, v, c, w2s, b2s, w3s, b3s, o_ref):
    s = jnp.zeros((1, OUT), jnp.float32)
    for b in range(3):
        t = jnp.dot(v[b], a[b], preferred_element_type=jnp.float32)
        u = jnp.dot(t, w2s[b], preferred_element_type=jnp.float32)
        u = u + jnp.sum(c[b]) * b2s[b][None, :]
        s = s + jnp.dot(u, w3s[b], preferred_element_type=jnp.float32)
        s = s + jnp.float32(N) * b3s[b][None, :]
    m = jnp.max(s)
    e = jnp.exp(s - m)
    o_ref[...] = e / jnp.sum(e)


def kernel(X0, X1, X2, L0, L1, L2, batch0, batch1, batch2,
           W01, b01, W02, b02, W03, b03,
           W11, b11, W12, b12, W13, b13,
           W21, b21, W22, b22, W23, b23):
    w1s = jnp.stack([W01, W11, W21])
    b1s = jnp.stack([b01, b11, b21])
    w2s = jnp.stack([W02, W12, W22])
    b2s = jnp.stack([b02, b12, b22])
    w3s = jnp.stack([W03, W13, W23])
    b3s = jnp.stack([b03, b13, b23])

    y1 = pl.pallas_call(
        _pre_body,
        out_shape=jax.ShapeDtypeStruct((3, N, H), jnp.float32),
    )(X0, X1, X2, w1s)

    lspec = pl.BlockSpec((BLK, N), lambda i: (i, 0))
    a, c = pl.pallas_call(
        _main_body,
        grid=(NBLK,),
        in_specs=[lspec, lspec, lspec,
                  pl.BlockSpec((3, N, H), lambda i: (0, 0, 0)),
                  pl.BlockSpec((3, H), lambda i: (0, 0))],
        out_specs=[pl.BlockSpec((3, BLK, H), lambda i: (0, i, 0)),
                   pl.BlockSpec((3, 1, N), lambda i: (0, 0, 0))],
        out_shape=[jax.ShapeDtypeStruct((3, N, H), jnp.float32),
                   jax.ShapeDtypeStruct((3, 1, N), jnp.float32)],
    )(L0, L1, L2, y1, b1s)

    v = pl.pallas_call(
        _v_body,
        grid=(NBLK,),
        in_specs=[lspec, lspec, lspec,
                  pl.BlockSpec((3, 1, BLK), lambda i: (0, 0, i))],
        out_specs=pl.BlockSpec((3, 1, N), lambda i: (0, 0, 0)),
        out_shape=jax.ShapeDtypeStruct((3, 1, N), jnp.float32),
    )(L0, L1, L2, c)

    out = pl.pallas_call(
        _final_body,
        out_shape=jax.ShapeDtypeStruct((1, OUT), jnp.float32),
    )(a, v, c, w2s, b2s, w3s, b3s)
    return out


# speedup vs baseline: 1.8608x; 1.3035x over previous
import jax
import jax.numpy as jnp
from jax.experimental import pallas as pl
from jax.experimental.pallas import tpu as pltpu

N = 4096
H = 32
OUT = 64
BLK = 1024
P = N // BLK
PA = P * P
PB = P * (P - 1) // 2
TOTAL = PA + PB


def _leaky(x):
    return jnp.where(x > 0, x, 0.01 * x)


def _tile_pq(t):
    qa = t // P
    ia = t % P
    pa = jnp.where(ia == P - 1, qa, ia + (ia >= qa).astype(jnp.int32))
    u = t - PA
    pb = 1 + (u >= 1).astype(jnp.int32) + (u >= 3).astype(jnp.int32)
    qb = u - pb * (pb - 1) // 2
    p = jnp.where(t < PA, pa, pb)
    q = jnp.where(t < PA, qa, qb)
    return p, q


def _pre_body(x0, x1, x2, w1s, y_ref):
    for b, xr in enumerate((x0, x1, x2)):
        y_ref[b] = jnp.dot(_leaky(xr[...]), w1s[b],
                           preferred_element_type=jnp.float32)


def _main_body(l0, l1, l2, y, b1s, w2s, b2s, w3s, b3s,
               o_ref, c_sc, v_sc, z_sc):
    t = pl.program_id(0)
    p, q = _tile_pq(t)
    ia = t % P
    phase_a = t < PA
    v_exec = jnp.logical_or(jnp.logical_not(phase_a),
                            jnp.logical_or(p < q, ia == P - 1))

    @pl.when(t == 0)
    def _():
        c_sc[...] = jnp.zeros_like(c_sc)
        v_sc[...] = jnp.zeros_like(v_sc)
        z_sc[...] = jnp.zeros_like(z_sc)

    for b, lr in enumerate((l0, l1, l2)):
        lb = lr[...]

        @pl.when(phase_a)
        def _(b=b, lb=lb):
            c_sc[b, :, pl.ds(q * BLK, BLK)] += jnp.sum(lb, axis=0,
                                                       keepdims=True)
            zc = jnp.dot(lb, y[b, pl.ds(q * BLK, BLK), :],
                         preferred_element_type=jnp.float32)
            z_sc[b, pl.ds(p * BLK, BLK), :] += zc

        @pl.when(v_exec)
        def _(b=b, lb=lb):
            cp = c_sc[b, :, pl.ds(p * BLK, BLK)]
            v_sc[b, :, pl.ds(q * BLK, BLK)] += jnp.dot(
                cp, lb, preferred_element_type=jnp.float32)

    @pl.when(t == TOTAL - 1)
    def _():
        s = jnp.zeros((1, OUT), jnp.float32)
        for b in range(3):
            a = _leaky(z_sc[b] + b1s[b][None, :])
            tb = jnp.dot(v_sc[b], a, preferred_element_type=jnp.float32)
            u = jnp.dot(tb, w2s[b], preferred_element_type=jnp.float32)
            u = u + jnp.sum(c_sc[b]) * b2s[b][None, :]
            s = s + jnp.dot(u, w3s[b], preferred_element_type=jnp.float32)
            s = s + jnp.float32(N) * b3s[b][None, :]
        m = jnp.max(s)
        e = jnp.exp(s - m)
        o_ref[...] = e / jnp.sum(e)


def kernel(X0, X1, X2, L0, L1, L2, batch0, batch1, batch2,
           W01, b01, W02, b02, W03, b03,
           W11, b11, W12, b12, W13, b13,
           W21, b21, W22, b22, W23, b23):
    w1s = jnp.stack([W01, W11, W21])
    b1s = jnp.stack([b01, b11, b21])
    w2s = jnp.stack([W02, W12, W22])
    b2s = jnp.stack([b02, b12, b22])
    w3s = jnp.stack([W03, W13, W23])
    b3s = jnp.stack([b03, b13, b23])

    y1 = pl.pallas_call(
        _pre_body,
        out_shape=jax.ShapeDtypeStruct((3, N, H), jnp.float32),
    )(X0, X1, X2, w1s)

    def lmap(t):
        p, q = _tile_pq(t)
        return p, q

    lspec = pl.BlockSpec((BLK, BLK), lmap)
    const3 = lambda shape: pl.BlockSpec(shape, lambda t: (0,) * len(shape))

    out = pl.pallas_call(
        _main_body,
        grid=(TOTAL,),
        in_specs=[lspec, lspec, lspec,
                  const3((3, N, H)),
                  const3((3, H)),
                  const3((3, H, H)),
                  const3((3, H)),
                  const3((3, H, OUT)),
                  const3((3, OUT))],
        out_specs=pl.BlockSpec((1, OUT), lambda t: (0, 0)),
        out_shape=jax.ShapeDtypeStruct((1, OUT), jnp.float32),
        scratch_shapes=[pltpu.VMEM((3, 1, N), jnp.float32),
                        pltpu.VMEM((3, 1, N), jnp.float32),
                        pltpu.VMEM((3, N, H), jnp.float32)],
    )(L0, L1, L2, y1, b1s, w2s, b2s, w3s, b3s)
    return out


# speedup vs baseline: 1.8990x; 1.0205x over previous
import jax
import jax.numpy as jnp
from jax.experimental import pallas as pl
from jax.experimental.pallas import tpu as pltpu

N = 4096
F = 128
H = 32
OUT = 64
BLK = 1024
P = N // BLK
PA = P * P
PB = P * (P - 1) // 2
TOTAL = PA + PB


def _leaky(x):
    return jnp.where(x > 0, x, 0.01 * x)


def _tile_pq(t):
    qa = t // P
    ia = t % P
    pa = jnp.where(ia == P - 1, qa, ia + (ia >= qa).astype(jnp.int32))
    u = t - PA
    pb = 1 + (u >= 1).astype(jnp.int32) + (u >= 3).astype(jnp.int32)
    qb = u - pb * (pb - 1) // 2
    p = jnp.where(t < PA, pa, pb)
    q = jnp.where(t < PA, qa, qb)
    return p, q


def _main_body(l0, l1, l2, x0, x1, x2, w1s, b1s, w2s, b2s, w3s, b3s,
               o_ref, c_sc, v_sc, z_sc, y_sc):
    t = pl.program_id(0)
    p, q = _tile_pq(t)
    ia = t % P
    phase_a = t < PA
    v_exec = jnp.logical_or(jnp.logical_not(phase_a),
                            jnp.logical_or(p < q, ia == P - 1))

    @pl.when(t == 0)
    def _():
        c_sc[...] = jnp.zeros_like(c_sc)
        v_sc[...] = jnp.zeros_like(v_sc)
        z_sc[...] = jnp.zeros_like(z_sc)
        for b, xr in enumerate((x0, x1, x2)):
            y_sc[b] = jnp.dot(_leaky(xr[...]), w1s[b],
                              preferred_element_type=jnp.float32)

    for b, lr in enumerate((l0, l1, l2)):
        lb = lr[...]

        @pl.when(phase_a)
        def _(b=b, lb=lb):
            c_sc[b, :, pl.ds(q * BLK, BLK)] += jnp.sum(lb, axis=0,
                                                       keepdims=True)
            zc = jnp.dot(lb, y_sc[b, pl.ds(q * BLK, BLK), :],
                         preferred_element_type=jnp.float32)
            z_sc[b, pl.ds(p * BLK, BLK), :] += zc

        @pl.when(v_exec)
        def _(b=b, lb=lb):
            cp = c_sc[b, :, pl.ds(p * BLK, BLK)]
            v_sc[b, :, pl.ds(q * BLK, BLK)] += jnp.dot(
                cp, lb, preferred_element_type=jnp.float32)

    @pl.when(t == TOTAL - 1)
    def _():
        s = jnp.zeros((1, OUT), jnp.float32)
        for b in range(3):
            a = _leaky(z_sc[b] + b1s[b][None, :])
            tb = jnp.dot(v_sc[b], a, preferred_element_type=jnp.float32)
            u = jnp.dot(tb, w2s[b], preferred_element_type=jnp.float32)
            u = u + jnp.sum(c_sc[b]) * b2s[b][None, :]
            s = s + jnp.dot(u, w3s[b], preferred_element_type=jnp.float32)
            s = s + jnp.float32(N) * b3s[b][None, :]
        m = jnp.max(s)
        e = jnp.exp(s - m)
        o_ref[...] = e / jnp.sum(e)


def kernel(X0, X1, X2, L0, L1, L2, batch0, batch1, batch2,
           W01, b01, W02, b02, W03, b03,
           W11, b11, W12, b12, W13, b13,
           W21, b21, W22, b22, W23, b23):
    w1s = jnp.stack([W01, W11, W21])
    b1s = jnp.stack([b01, b11, b21])
    w2s = jnp.stack([W02, W12, W22])
    b2s = jnp.stack([b02, b12, b22])
    w3s = jnp.stack([W03, W13, W23])
    b3s = jnp.stack([b03, b13, b23])

    lspec = pl.BlockSpec((BLK, BLK), _tile_pq)
    const = lambda shape: pl.BlockSpec(shape, lambda t: (0,) * len(shape))

    out = pl.pallas_call(
        _main_body,
        grid=(TOTAL,),
        in_specs=[lspec, lspec, lspec,
                  const((N, F)), const((N, F)), const((N, F)),
                  const((3, F, H)),
                  const((3, H)),
                  const((3, H, H)),
                  const((3, H)),
                  const((3, H, OUT)),
                  const((3, OUT))],
        out_specs=pl.BlockSpec((1, OUT), lambda t: (0, 0)),
        out_shape=jax.ShapeDtypeStruct((1, OUT), jnp.float32),
        scratch_shapes=[pltpu.VMEM((3, 1, N), jnp.float32),
                        pltpu.VMEM((3, 1, N), jnp.float32),
                        pltpu.VMEM((3, N, H), jnp.float32),
                        pltpu.VMEM((3, N, H), jnp.float32)],
    )(L0, L1, L2, X0, X1, X2, w1s, b1s, w2s, b2s, w3s, b3s)
    return out


# speedup vs baseline: 1.8990x; 1.0000x over previous
import jax
import jax.numpy as jnp
from jax.experimental import pallas as pl
from jax.experimental.pallas import tpu as pltpu

N = 4096
F = 128
H = 32
OUT = 64
BLK = 1024
P = N // BLK
PA = P * P
PB = 5
TOTAL = PA + PB


def _leaky(x):
    return jnp.where(x > 0, x, 0.01 * x)


def _tile_pq(t):
    qa = t // P
    ia = t % P
    pa = jnp.where(ia == P - 1, qa, ia + (ia >= qa).astype(jnp.int32))
    u = t - PA
    pb = 2 + (u >= 2).astype(jnp.int32)
    qb = jnp.where(u < 2, u, u - 2)
    p = jnp.where(t < PA, pa, pb)
    q = jnp.where(t < PA, qa, qb)
    return p, q


def _pre_body(x0, x1, x2, w1s, y_ref):
    for b, xr in enumerate((x0, x1, x2)):
        y_ref[b] = jnp.dot(_leaky(xr[...]), w1s[b],
                           preferred_element_type=jnp.float32)


def _main_body(l0, l1, l2, y, b1s, w2s, b2s, w3s, b3s,
               o_ref, c_sc, v_sc, z_sc, h_sc):
    t = pl.program_id(0)
    p, q = _tile_pq(t)
    ia = t % P
    phase_a = t < PA
    v_exec = jnp.logical_or(jnp.logical_not(phase_a),
                            jnp.logical_or(p < q, ia == P - 1))

    @pl.when(t == 0)
    def _():
        c_sc[...] = jnp.zeros_like(c_sc)
        v_sc[...] = jnp.zeros_like(v_sc)
        z_sc[...] = jnp.zeros_like(z_sc)

    for b, lr in enumerate((l0, l1, l2)):
        lb = lr[...]

        @pl.when(phase_a)
        def _(b=b, lb=lb):
            c_sc[b, :, pl.ds(q * BLK, BLK)] += jnp.sum(lb, axis=0,
                                                       keepdims=True)
            zc = jnp.dot(lb, y[b, pl.ds(q * BLK, BLK), :],
                         preferred_element_type=jnp.float32)
            z_sc[b, pl.ds(p * BLK, BLK), :] += zc

        @pl.when(t == 0)
        def _(b=b, lb=lb):
            h_sc[b] = lb

        @pl.when(v_exec)
        def _(b=b, lb=lb):
            cp = c_sc[b, :, pl.ds(p * BLK, BLK)]
            v_sc[b, :, pl.ds(q * BLK, BLK)] += jnp.dot(
                cp, lb, preferred_element_type=jnp.float32)

        @pl.when(t == PA)
        def _(b=b):
            v_sc[b, :, pl.ds(0, BLK)] += jnp.dot(
                c_sc[b, :, pl.ds(1 * BLK, BLK)], h_sc[b],
                preferred_element_type=jnp.float32)

    @pl.when(t == TOTAL - 1)
    def _():
        s = jnp.zeros((1, OUT), jnp.float32)
        for b in range(3):
            a = _leaky(z_sc[b] + b1s[b][None, :])
            tb = jnp.dot(v_sc[b], a, preferred_element_type=jnp.float32)
            u = jnp.dot(tb, w2s[b], preferred_element_type=jnp.float32)
            u = u + jnp.sum(c_sc[b]) * b2s[b][None, :]
            s = s + jnp.dot(u, w3s[b], preferred_element_type=jnp.float32)
            s = s + jnp.float32(N) * b3s[b][None, :]
        m = jnp.max(s)
        e = jnp.exp(s - m)
        o_ref[...] = e / jnp.sum(e)


def kernel(X0, X1, X2, L0, L1, L2, batch0, batch1, batch2,
           W01, b01, W02, b02, W03, b03,
           W11, b11, W12, b12, W13, b13,
           W21, b21, W22, b22, W23, b23):
    w1s = jnp.stack([W01, W11, W21])
    b1s = jnp.stack([b01, b11, b21])
    w2s = jnp.stack([W02, W12, W22])
    b2s = jnp.stack([b02, b12, b22])
    w3s = jnp.stack([W03, W13, W23])
    b3s = jnp.stack([b03, b13, b23])

    y1 = pl.pallas_call(
        _pre_body,
        out_shape=jax.ShapeDtypeStruct((3, N, H), jnp.float32),
    )(X0, X1, X2, w1s)

    lspec = pl.BlockSpec((BLK, BLK), _tile_pq)
    const = lambda shape: pl.BlockSpec(shape, lambda t: (0,) * len(shape))

    out = pl.pallas_call(
        _main_body,
        grid=(TOTAL,),
        in_specs=[lspec, lspec, lspec,
                  const((3, N, H)),
                  const((3, H)),
                  const((3, H, H)),
                  const((3, H)),
                  const((3, H, OUT)),
                  const((3, OUT))],
        out_specs=pl.BlockSpec((1, OUT), lambda t: (0, 0)),
        out_shape=jax.ShapeDtypeStruct((1, OUT), jnp.float32),
        scratch_shapes=[pltpu.VMEM((3, 1, N), jnp.float32),
                        pltpu.VMEM((3, 1, N), jnp.float32),
                        pltpu.VMEM((3, N, H), jnp.float32),
                        pltpu.VMEM((3, BLK, BLK), jnp.float32)],
    )(L0, L1, L2, y1, b1s, w2s, b2s, w3s, b3s)
    return out
